# Initial kernel scaffold; baseline (speedup 1.0000x reference)
import jax
import jax.numpy as jnp
from jax.experimental import pallas as pl


def kernel(input_ids, attention_mask, embed_table):
    raise NotImplementedError("write your pallas kernel here")


# speedup vs baseline: 1.4739x; 1.4739x over previous
import functools

import jax
import jax.numpy as jnp
from jax import lax
from jax.experimental import pallas as pl
from jax.experimental.pallas import tpu as pltpu
from jax.experimental.pallas import tpu_sc as plsc

_NC = 2
_NS = 16
_NW = _NC * _NS


def _gather_call(idx3, table, n_tok, hidden):
    n_chunks, chunk = idx3.shape[1], idx3.shape[2]
    per_w = n_chunks * chunk

    mesh = plsc.VectorSubcoreMesh(core_axis_name="c", subcore_axis_name="s")

    @functools.partial(
        pl.kernel,
        mesh=mesh,
        out_type=jax.ShapeDtypeStruct((n_tok, hidden), jnp.float32),
        scratch_types=[
            pltpu.VMEM((n_chunks, chunk), jnp.int32),
            pltpu.VMEM((chunk, hidden), jnp.float32),
            pltpu.VMEM((chunk, hidden), jnp.float32),
            pltpu.SemaphoreType.DMA,
            pltpu.SemaphoreType.DMA,
            pltpu.SemaphoreType.DMA,
            pltpu.SemaphoreType.DMA,
        ],
    )
    def body(idx_hbm, table_hbm, out_hbm, idx_v, buf0, buf1,
             gsem0, gsem1, wsem0, wsem1):
        wid = lax.axis_index("s") * _NC + lax.axis_index("c")
        base = wid * per_w
        pltpu.sync_copy(idx_hbm.at[wid], idx_v)
        bufs = (buf0, buf1)
        gsems = (gsem0, gsem1)
        wsems = (wsem0, wsem1)
        g = [None] * n_chunks
        w = [None] * n_chunks
        g[0] = pltpu.async_copy(table_hbm.at[idx_v.at[0]], buf0, gsem0)
        for j in range(n_chunks):
            if j + 1 < n_chunks:
                if j >= 1:
                    w[j - 1].wait()
                g[j + 1] = pltpu.async_copy(
                    table_hbm.at[idx_v.at[j + 1]], bufs[(j + 1) % 2],
                    gsems[(j + 1) % 2])
            g[j].wait()
            w[j] = pltpu.async_copy(
                bufs[j % 2], out_hbm.at[pl.ds(base + j * chunk, chunk)],
                wsems[j % 2])
        w[n_chunks - 2].wait()
        w[n_chunks - 1].wait()

    return body


def _mask_body(t, br, b, am_ref, attn_ref, pos_ref):
    r0 = pl.program_id(1) * br
    rows = r0 + lax.broadcasted_iota(jnp.int32, (br, t), 0)
    cols = lax.broadcasted_iota(jnp.int32, (br, t), 1)
    pad = (am_ref[0] == 0)
    bad = (cols > rows) | pad
    attn_ref[0, 0] = jnp.where(bad, jnp.float32(-jnp.inf), jnp.float32(0.0))
    pos_ref[...] = lax.broadcasted_iota(jnp.int32, (3, b, t), 2)


def kernel(input_ids, attention_mask, embed_table):
    b, t = input_ids.shape
    vocab, hidden = embed_table.shape
    n_tok = b * t

    chunk = 32
    per_w = n_tok // _NW
    n_chunks = per_w // chunk
    idx3 = input_ids.reshape(_NW, n_chunks, chunk)
    flat = _gather_call(idx3, embed_table, n_tok, hidden)(idx3, embed_table)
    hidden_out = flat.reshape(b, t, hidden)

    br = 256
    grid = (b, t // br)
    attn, pos = pl.pallas_call(
        functools.partial(_mask_body, t, br, b),
        grid=grid,
        in_specs=[pl.BlockSpec((1, 1, t), lambda i, j: (i, 0, 0))],
        out_specs=[
            pl.BlockSpec((1, 1, br, t), lambda i, j: (i, 0, j, 0)),
            pl.BlockSpec((3, b, t), lambda i, j: (0, 0, 0)),
        ],
        out_shape=[
            jax.ShapeDtypeStruct((b, 1, t, t), jnp.float32),
            jax.ShapeDtypeStruct((3, b, t), jnp.int32),
        ],
    )(attention_mask.reshape(b, 1, t))

    return hidden_out, attn, pos


# speedup vs baseline: 1.5234x; 1.0336x over previous
import functools

import jax
import jax.numpy as jnp
from jax import lax
from jax.experimental import pallas as pl
from jax.experimental.pallas import tpu as pltpu
from jax.experimental.pallas import tpu_sc as plsc

_NC = 2
_NS = 16
_NW = _NC * _NS


def _gather_call(b, t, hidden, n_chunks, chunk):
    per_w = n_chunks * chunk
    w_per_row = t // per_w

    mesh = plsc.VectorSubcoreMesh(core_axis_name="c", subcore_axis_name="s")

    @functools.partial(
        pl.kernel,
        mesh=mesh,
        out_type=jax.ShapeDtypeStruct((b, t, hidden), jnp.float32),
        scratch_types=[
            pltpu.VMEM((n_chunks, chunk), jnp.int32),
            pltpu.VMEM((chunk, hidden), jnp.float32),
            pltpu.VMEM((chunk, hidden), jnp.float32),
            pltpu.SemaphoreType.DMA,
            pltpu.SemaphoreType.DMA,
            pltpu.SemaphoreType.DMA,
            pltpu.SemaphoreType.DMA,
        ],
    )
    def body(idx_hbm, table_hbm, out_hbm, idx_v, buf0, buf1,
             gsem0, gsem1, wsem0, wsem1):
        wid = lax.axis_index("s") * _NC + lax.axis_index("c")
        row = wid // w_per_row
        col0 = (wid % w_per_row) * per_w
        for j in range(n_chunks):
            pltpu.sync_copy(idx_hbm.at[row, pl.ds(col0 + j * chunk, chunk)],
                            idx_v.at[j])
        bufs = (buf0, buf1)
        gsems = (gsem0, gsem1)
        wsems = (wsem0, wsem1)
        g = [None] * n_chunks
        w = [None] * n_chunks
        g[0] = pltpu.async_copy(table_hbm.at[idx_v.at[0]], buf0, gsem0)
        for j in range(n_chunks):
            if j + 1 < n_chunks:
                if j >= 1:
                    w[j - 1].wait()
                g[j + 1] = pltpu.async_copy(
                    table_hbm.at[idx_v.at[j + 1]], bufs[(j + 1) % 2],
                    gsems[(j + 1) % 2])
            g[j].wait()
            w[j] = pltpu.async_copy(
                bufs[j % 2],
                out_hbm.at[row, pl.ds(col0 + j * chunk, chunk)],
                wsems[j % 2])
        w[n_chunks - 2].wait()
        w[n_chunks - 1].wait()

    return body


def _mask_body(t, br, b, am_ref, attn_ref, pos_ref):
    i = pl.program_id(0)
    r0 = pl.program_id(1) * br
    rows = r0 + lax.broadcasted_iota(jnp.int32, (br, t), 0)
    cols = lax.broadcasted_iota(jnp.int32, (br, t), 1)
    pad = (am_ref[pl.ds(i, 1), :] == 0)
    bad = (cols > rows) | pad
    attn_ref[0, 0] = jnp.where(bad, jnp.float32(-jnp.inf), jnp.float32(0.0))
    pos_ref[...] = lax.broadcasted_iota(jnp.int32, (3, b, t), 2)


def kernel(input_ids, attention_mask, embed_table):
    b, t = input_ids.shape
    vocab, hidden = embed_table.shape
    n_tok = b * t

    chunk = 32
    per_w = n_tok // _NW
    n_chunks = per_w // chunk
    hidden_out = _gather_call(b, t, hidden, n_chunks, chunk)(
        input_ids, embed_table)

    br = 256
    grid = (b, t // br)
    attn, pos = pl.pallas_call(
        functools.partial(_mask_body, t, br, b),
        grid=grid,
        in_specs=[pl.BlockSpec((b, t), lambda i, j: (0, 0))],
        out_specs=[
            pl.BlockSpec((1, 1, br, t), lambda i, j: (i, 0, j, 0)),
            pl.BlockSpec((3, b, t), lambda i, j: (0, 0, 0)),
        ],
        out_shape=[
            jax.ShapeDtypeStruct((b, 1, t, t), jnp.float32),
            jax.ShapeDtypeStruct((3, b, t), jnp.int32),
        ],
    )(attention_mask)

    return hidden_out, attn, pos


# speedup vs baseline: 1.5669x; 1.0286x over previous
import functools

import jax
import jax.numpy as jnp
from jax import lax
from jax.experimental import pallas as pl
from jax.experimental.pallas import tpu as pltpu
from jax.experimental.pallas import tpu_sc as plsc

_NC = 2
_NS = 16
_NW = _NC * _NS


def _gather_call(b, t, hidden, n_chunks, chunk):
    per_w = n_chunks * chunk
    w_per_row = t // per_w

    mesh = plsc.VectorSubcoreMesh(core_axis_name="c", subcore_axis_name="s")

    @functools.partial(
        pl.kernel,
        mesh=mesh,
        out_type=jax.ShapeDtypeStruct((b, t, hidden), jnp.float32),
        scratch_types=[
            pltpu.VMEM((per_w,), jnp.int32),
            pltpu.VMEM((chunk, hidden), jnp.float32),
            pltpu.VMEM((chunk, hidden), jnp.float32),
            pltpu.VMEM((chunk, hidden), jnp.float32),
            pltpu.SemaphoreType.DMA,
            pltpu.SemaphoreType.DMA,
            pltpu.SemaphoreType.DMA,
            pltpu.SemaphoreType.DMA,
            pltpu.SemaphoreType.DMA,
            pltpu.SemaphoreType.DMA,
        ],
    )
    def body(idx_hbm, table_hbm, out_hbm, idx_v, buf0, buf1, buf2,
             gsem0, gsem1, gsem2, wsem0, wsem1, wsem2):
        wid = lax.axis_index("s") * _NC + lax.axis_index("c")
        row = wid // w_per_row
        col0 = (wid % w_per_row) * per_w
        pltpu.sync_copy(idx_hbm.at[row, pl.ds(col0, per_w)], idx_v)
        bufs = (buf0, buf1, buf2)
        gsems = (gsem0, gsem1, gsem2)
        wsems = (wsem0, wsem1, wsem2)
        g = [None] * n_chunks
        w = [None] * n_chunks
        for j in range(min(3, n_chunks)):
            g[j] = pltpu.async_copy(
                table_hbm.at[idx_v.at[pl.ds(j * chunk, chunk)]], bufs[j],
                gsems[j])
        for j in range(n_chunks):
            if j >= 2 and (j + 1) < n_chunks:
                w[j - 2].wait()
                g[j + 1] = pltpu.async_copy(
                    table_hbm.at[idx_v.at[pl.ds((j + 1) * chunk, chunk)]],
                    bufs[(j + 1) % 3], gsems[(j + 1) % 3])
            g[j].wait()
            w[j] = pltpu.async_copy(
                bufs[j % 3],
                out_hbm.at[row, pl.ds(col0 + j * chunk, chunk)],
                wsems[j % 3])
        for j in range(max(0, n_chunks - 3), n_chunks):
            w[j].wait()

    return body


def _mask_body(t, br, b, am_ref, attn_ref, pos_ref):
    i = pl.program_id(0)
    r0 = pl.program_id(1) * br
    rows = r0 + lax.broadcasted_iota(jnp.int32, (br, t), 0)
    cols = lax.broadcasted_iota(jnp.int32, (br, t), 1)
    pad = (am_ref[pl.ds(i, 1), :] == 0)
    bad = (cols > rows) | pad
    attn_ref[0, 0] = jnp.where(bad, jnp.float32(-jnp.inf), jnp.float32(0.0))
    pos_ref[...] = lax.broadcasted_iota(jnp.int32, (3, b, t), 2)


def kernel(input_ids, attention_mask, embed_table):
    b, t = input_ids.shape
    vocab, hidden = embed_table.shape
    n_tok = b * t

    chunk = 32
    per_w = n_tok // _NW
    n_chunks = per_w // chunk
    hidden_out = _gather_call(b, t, hidden, n_chunks, chunk)(
        input_ids, embed_table)

    br = 512
    grid = (b, t // br)
    attn, pos = pl.pallas_call(
        functools.partial(_mask_body, t, br, b),
        grid=grid,
        in_specs=[pl.BlockSpec((b, t), lambda i, j: (0, 0))],
        out_specs=[
            pl.BlockSpec((1, 1, br, t), lambda i, j: (i, 0, j, 0)),
            pl.BlockSpec((3, b, t), lambda i, j: (0, 0, 0)),
        ],
        out_shape=[
            jax.ShapeDtypeStruct((b, 1, t, t), jnp.float32),
            jax.ShapeDtypeStruct((3, b, t), jnp.int32),
        ],
    )(attention_mask)

    return hidden_out, attn, pos
